# baseline (device time: 112295 ns/iter reference)
import contextlib
import os

import jax
import jax.numpy as jnp
from jax import lax
from jax.experimental import pallas as pl
from jax.experimental.pallas import tpu as pltpu

if os.environ.get("KERNEL_SCOPES") == "1":
    _scope = jax.named_scope
else:
    def _scope(name):
        return contextlib.nullcontext()

M, N = 8192, 1024
QROWS = M // 4
CH = 512
KQ = QROWS // CH
KZ = KQ - 1


def kernel(x):
    dummy = lax.broadcast(x[0, 0].astype(jnp.bfloat16), (M, N))

    def body(x_hbm, dummy_hbm, out_hbm, loc32, rem, sendq, out_stage,
             in_sems, bulk_sems, send_sems, xe_send, x_recv, y_recv,
             za_recv, zb_recv, xe_recv, out_sems):
        del dummy_hbm
        my_x = lax.axis_index("x")
        my_y = lax.axis_index("y")
        my_z = lax.axis_index("z")
        px = (1 - my_x, my_y, my_z)
        py = (my_x, 1 - my_y, my_z)
        pz = (my_x, my_y, 1 - my_z)

        q_own = 2 * my_y + my_z
        q_y = 2 * (1 - my_y) + my_z
        q_za = 2 * my_y + (1 - my_z)
        q_zb = 2 * (1 - my_y) + (1 - my_z)

        def rows(q, k=0, nrows=CH):
            return pl.ds(q * QROWS + k * CH, nrows)

        with _scope("barrier"):
            barrier_sem = pltpu.get_barrier_semaphore()
            for nbr in (px, py, pz):
                pl.semaphore_signal(
                    barrier_sem, inc=1, device_id=nbr,
                    device_id_type=pl.DeviceIdType.MESH,
                )
            pl.semaphore_wait(barrier_sem, 3)

        in_cps = []
        for k in range(KQ):
            cp = pltpu.make_async_copy(
                x_hbm.at[rows(q_own, k), :],
                loc32.at[rows(q_own, k), :],
                in_sems.at[k],
            )
            cp.start()
            in_cps.append(cp)

        x_rdmas = []
        for k in range(KQ):
            with _scope(f"inject#k={k}"):
                in_cps[k].wait()
                sendq[pl.ds(k * CH, CH), :] = loc32[rows(q_own, k), :].astype(
                    jnp.bfloat16)
                rdma = pltpu.make_async_remote_copy(
                    src_ref=sendq.at[pl.ds(k * CH, CH), :],
                    dst_ref=rem.at[rows(q_own, k), :],
                    send_sem=in_sems.at[k],
                    recv_sem=x_recv.at[k],
                    device_id=px,
                    device_id_type=pl.DeviceIdType.MESH,
                )
                rdma.start()
                x_rdmas.append(rdma)

        bulk_cps = []
        for i, q in enumerate((q_y, q_za, q_zb)):
            cp = pltpu.make_async_copy(
                x_hbm.at[rows(q, 0, QROWS), :],
                loc32.at[rows(q, 0, QROWS), :],
                bulk_sems.at[i],
            )
            cp.start()
            bulk_cps.append(cp)

        xe_rdmas = []
        for i, q in enumerate((q_za, q_zb)):
            with _scope(f"xe_inject#i={i}"):
                bulk_cps[1 + i].wait()
                slot = pl.ds((KQ + i) * CH, CH)
                sendq[slot, :] = loc32[rows(q, KZ), :].astype(jnp.bfloat16)
                rdma = pltpu.make_async_remote_copy(
                    src_ref=sendq.at[slot, :],
                    dst_ref=rem.at[rows(q, KZ), :],
                    send_sem=xe_send.at[i],
                    recv_sem=xe_recv.at[i],
                    device_id=px,
                    device_id_type=pl.DeviceIdType.MESH,
                )
                rdma.start()
                xe_rdmas.append(rdma)

        out_cps = [None, None]
        emit_count = [0]

        def emit(q, k):
            i = emit_count[0]
            slot = i % 2
            if out_cps[slot] is not None:
                out_cps[slot].wait()
            out_stage[slot] = (
                loc32[rows(q, k), :].astype(jnp.bfloat16) + rem[rows(q, k), :]
            )
            cp = pltpu.make_async_copy(
                out_stage.at[slot], out_hbm.at[rows(q, k), :],
                out_sems.at[slot],
            )
            cp.start()
            out_cps[slot] = cp
            emit_count[0] += 1

        y_rdmas, za_rdmas = [], []
        for k in range(KQ):
          with _scope(f"xhop#k={k}"):
            x_rdmas[k].wait_recv()
            ry = pltpu.make_async_remote_copy(
                src_ref=rem.at[rows(q_own, k), :],
                dst_ref=rem.at[rows(q_own, k), :],
                send_sem=send_sems.at[0],
                recv_sem=y_recv.at[k],
                device_id=py,
                device_id_type=pl.DeviceIdType.MESH,
            )
            ry.start()
            y_rdmas.append(ry)
            if k < KZ:
                rza = pltpu.make_async_remote_copy(
                    src_ref=rem.at[rows(q_own, k), :],
                    dst_ref=rem.at[rows(q_own, k), :],
                    send_sem=send_sems.at[1],
                    recv_sem=za_recv.at[k],
                    device_id=pz,
                    device_id_type=pl.DeviceIdType.MESH,
                )
                rza.start()
                za_rdmas.append(rza)
            emit(q_own, k)

        bulk_cps[0].wait()
        zb_rdmas = []
        for k in range(KQ):
          with _scope(f"yhop#k={k}"):
            ry_in = pltpu.make_async_remote_copy(
                src_ref=rem.at[rows(q_y, k), :],
                dst_ref=rem.at[rows(q_y, k), :],
                send_sem=send_sems.at[0],
                recv_sem=y_recv.at[k],
                device_id=py,
                device_id_type=pl.DeviceIdType.MESH,
            )
            ry_in.wait_recv()
            if k < KZ:
                rzb = pltpu.make_async_remote_copy(
                    src_ref=rem.at[rows(q_y, k), :],
                    dst_ref=rem.at[rows(q_y, k), :],
                    send_sem=send_sems.at[2],
                    recv_sem=zb_recv.at[k],
                    device_id=pz,
                    device_id_type=pl.DeviceIdType.MESH,
                )
                rzb.start()
                zb_rdmas.append(rzb)
            emit(q_y, k)

        for k in range(KZ):
            with _scope(f"za#k={k}"):
                za_rdmas[k].wait_recv()
                emit(q_za, k)
        with _scope("za_xe"):
            xe_rdmas[0].wait_recv()
            emit(q_za, KZ)
        for k in range(KZ):
            with _scope(f"zb#k={k}"):
                zb_rdmas[k].wait_recv()
                emit(q_zb, k)
        with _scope("zb_xe"):
            xe_rdmas[1].wait_recv()
            emit(q_zb, KZ)

        with _scope("drain"):
            for cp in out_cps:
                if cp is not None:
                    cp.wait()
            for k in range(KQ):
                x_rdmas[k].wait_send()
                y_rdmas[k].wait_send()
            for k in range(KZ):
                za_rdmas[k].wait_send()
                zb_rdmas[k].wait_send()
            xe_rdmas[0].wait_send()
            xe_rdmas[1].wait_send()

    return pl.pallas_call(
        body,
        out_shape=jax.ShapeDtypeStruct((M, N), jnp.bfloat16),
        in_specs=[pl.BlockSpec(memory_space=pl.ANY),
                  pl.BlockSpec(memory_space=pl.ANY)],
        out_specs=pl.BlockSpec(memory_space=pl.ANY),
        input_output_aliases={1: 0},
        scratch_shapes=[
            pltpu.VMEM((M, N), jnp.float32),
            pltpu.VMEM((M, N), jnp.bfloat16),
            pltpu.VMEM((QROWS + 2 * CH, N), jnp.bfloat16),
            pltpu.VMEM((2, CH, N), jnp.bfloat16),
            pltpu.SemaphoreType.DMA((KQ,)),
            pltpu.SemaphoreType.DMA((3,)),
            pltpu.SemaphoreType.DMA((3,)),
            pltpu.SemaphoreType.DMA((2,)),
            pltpu.SemaphoreType.DMA((KQ,)),
            pltpu.SemaphoreType.DMA((KQ,)),
            pltpu.SemaphoreType.DMA((KZ,)),
            pltpu.SemaphoreType.DMA((KZ,)),
            pltpu.SemaphoreType.DMA((2,)),
            pltpu.SemaphoreType.DMA((2,)),
        ],
        compiler_params=pltpu.CompilerParams(
            collective_id=0, vmem_limit_bytes=60 * 1024 * 1024),
    )(x, dummy)


# device time: 103454 ns/iter; 1.0855x vs baseline; 1.0855x over previous
import contextlib
import os

import jax
import jax.numpy as jnp
from jax import lax
from jax.experimental import pallas as pl
from jax.experimental.pallas import tpu as pltpu

if os.environ.get("KERNEL_SCOPES") == "1":
    _scope = jax.named_scope
else:
    def _scope(name):
        return contextlib.nullcontext()

M, N = 8192, 1024
QROWS = M // 4
CH = 256
KQ = QROWS // CH
XD = max(1, KQ // 4)
KZ = KQ - XD


def kernel(x):
    def body(x_hbm, out_hbm, loc32, rem, sendq, out_stage,
             in_sems, bulk_sems, send_sems, xe_send, x_recv, y_recv,
             za_recv, zb_recv, xe_recv, out_sems):
        my_x = lax.axis_index("x")
        my_y = lax.axis_index("y")
        my_z = lax.axis_index("z")
        px = (1 - my_x, my_y, my_z)
        py = (my_x, 1 - my_y, my_z)
        pz = (my_x, my_y, 1 - my_z)

        q_own = 2 * my_y + my_z
        q_y = 2 * (1 - my_y) + my_z
        q_za = 2 * my_y + (1 - my_z)
        q_zb = 2 * (1 - my_y) + (1 - my_z)

        def rows(q, k=0, nrows=CH):
            return pl.ds(q * QROWS + k * CH, nrows)

        with _scope("barrier"):
            barrier_sem = pltpu.get_barrier_semaphore()
            for nbr in (px, py, pz):
                pl.semaphore_signal(
                    barrier_sem, inc=1, device_id=nbr,
                    device_id_type=pl.DeviceIdType.MESH,
                )
            pl.semaphore_wait(barrier_sem, 3)

        in_cps = []
        for k in range(KQ):
            cp = pltpu.make_async_copy(
                x_hbm.at[rows(q_own, k), :],
                loc32.at[rows(q_own, k), :],
                in_sems.at[k],
            )
            cp.start()
            in_cps.append(cp)

        x_rdmas = []
        for k in range(KQ):
            with _scope(f"inject#k={k}"):
                in_cps[k].wait()
                sendq[pl.ds(k * CH, CH), :] = loc32[rows(q_own, k), :].astype(
                    jnp.bfloat16)
                rdma = pltpu.make_async_remote_copy(
                    src_ref=sendq.at[pl.ds(k * CH, CH), :],
                    dst_ref=rem.at[rows(q_own, k), :],
                    send_sem=in_sems.at[k],
                    recv_sem=x_recv.at[k],
                    device_id=px,
                    device_id_type=pl.DeviceIdType.MESH,
                )
                rdma.start()
                x_rdmas.append(rdma)

        bulk_cps = []
        for i, q in enumerate((q_y, q_za, q_zb)):
            cp = pltpu.make_async_copy(
                x_hbm.at[rows(q, 0, QROWS), :],
                loc32.at[rows(q, 0, QROWS), :],
                bulk_sems.at[i],
            )
            cp.start()
            bulk_cps.append(cp)

        xe_rdmas = {}
        idx = 0
        for qi, q in enumerate((q_za, q_zb)):
            with _scope(f"xe_inject#q={qi}"):
                bulk_cps[1 + qi].wait()
                for j in range(KZ, KQ):
                    slot = pl.ds((KQ + idx) * CH, CH)
                    sendq[slot, :] = loc32[rows(q, j), :].astype(jnp.bfloat16)
                    rdma = pltpu.make_async_remote_copy(
                        src_ref=sendq.at[slot, :],
                        dst_ref=rem.at[rows(q, j), :],
                        send_sem=xe_send.at[idx],
                        recv_sem=xe_recv.at[idx],
                        device_id=px,
                        device_id_type=pl.DeviceIdType.MESH,
                    )
                    rdma.start()
                    xe_rdmas[(qi, j)] = rdma
                    idx += 1

        out_cps = [None, None]
        emit_count = [0]

        def emit(q, k):
            i = emit_count[0]
            slot = i % 2
            if out_cps[slot] is not None:
                out_cps[slot].wait()
            out_stage[slot] = (
                loc32[rows(q, k), :].astype(jnp.bfloat16) + rem[rows(q, k), :]
            )
            cp = pltpu.make_async_copy(
                out_stage.at[slot], out_hbm.at[rows(q, k), :],
                out_sems.at[slot],
            )
            cp.start()
            out_cps[slot] = cp
            emit_count[0] += 1

        y_rdmas, za_rdmas = [], []
        for k in range(KQ):
          with _scope(f"xhop#k={k}"):
            x_rdmas[k].wait_recv()
            ry = pltpu.make_async_remote_copy(
                src_ref=rem.at[rows(q_own, k), :],
                dst_ref=rem.at[rows(q_own, k), :],
                send_sem=send_sems.at[0],
                recv_sem=y_recv.at[k],
                device_id=py,
                device_id_type=pl.DeviceIdType.MESH,
            )
            ry.start()
            y_rdmas.append(ry)
            if k < KZ:
                rza = pltpu.make_async_remote_copy(
                    src_ref=rem.at[rows(q_own, k), :],
                    dst_ref=rem.at[rows(q_own, k), :],
                    send_sem=send_sems.at[1],
                    recv_sem=za_recv.at[k],
                    device_id=pz,
                    device_id_type=pl.DeviceIdType.MESH,
                )
                rza.start()
                za_rdmas.append(rza)
            emit(q_own, k)

        bulk_cps[0].wait()
        zb_rdmas = []
        for k in range(KQ):
          with _scope(f"yhop#k={k}"):
            ry_in = pltpu.make_async_remote_copy(
                src_ref=rem.at[rows(q_y, k), :],
                dst_ref=rem.at[rows(q_y, k), :],
                send_sem=send_sems.at[0],
                recv_sem=y_recv.at[k],
                device_id=py,
                device_id_type=pl.DeviceIdType.MESH,
            )
            ry_in.wait_recv()
            if k < KZ:
                rzb = pltpu.make_async_remote_copy(
                    src_ref=rem.at[rows(q_y, k), :],
                    dst_ref=rem.at[rows(q_y, k), :],
                    send_sem=send_sems.at[2],
                    recv_sem=zb_recv.at[k],
                    device_id=pz,
                    device_id_type=pl.DeviceIdType.MESH,
                )
                rzb.start()
                zb_rdmas.append(rzb)
            emit(q_y, k)

        for k in range(KZ):
            with _scope(f"za#k={k}"):
                za_rdmas[k].wait_recv()
                emit(q_za, k)
        with _scope("za_xe"):
            for j in range(KZ, KQ):
                xe_rdmas[(0, j)].wait_recv()
                emit(q_za, j)
        for k in range(KZ):
            with _scope(f"zb#k={k}"):
                zb_rdmas[k].wait_recv()
                emit(q_zb, k)
        with _scope("zb_xe"):
            for j in range(KZ, KQ):
                xe_rdmas[(1, j)].wait_recv()
                emit(q_zb, j)

        with _scope("drain"):
            for cp in out_cps:
                if cp is not None:
                    cp.wait()
            for k in range(KQ):
                x_rdmas[k].wait_send()
                y_rdmas[k].wait_send()
            for k in range(KZ):
                za_rdmas[k].wait_send()
                zb_rdmas[k].wait_send()
            for r in xe_rdmas.values():
                r.wait_send()

    return pl.pallas_call(
        body,
        out_shape=jax.ShapeDtypeStruct((M, N), jnp.bfloat16),
        in_specs=[pl.BlockSpec(memory_space=pl.ANY)],
        out_specs=pl.BlockSpec(memory_space=pl.ANY),
        scratch_shapes=[
            pltpu.VMEM((M, N), jnp.float32),
            pltpu.VMEM((M, N), jnp.bfloat16),
            pltpu.VMEM((QROWS + 2 * XD * CH, N), jnp.bfloat16),
            pltpu.VMEM((2, CH, N), jnp.bfloat16),
            pltpu.SemaphoreType.DMA((KQ,)),
            pltpu.SemaphoreType.DMA((3,)),
            pltpu.SemaphoreType.DMA((3,)),
            pltpu.SemaphoreType.DMA((2 * XD,)),
            pltpu.SemaphoreType.DMA((KQ,)),
            pltpu.SemaphoreType.DMA((KQ,)),
            pltpu.SemaphoreType.DMA((KZ,)),
            pltpu.SemaphoreType.DMA((KZ,)),
            pltpu.SemaphoreType.DMA((2 * XD,)),
            pltpu.SemaphoreType.DMA((2,)),
        ],
        compiler_params=pltpu.CompilerParams(
            collective_id=0, vmem_limit_bytes=60 * 1024 * 1024),
    )(x)


# device time: 96595 ns/iter; 1.1625x vs baseline; 1.0710x over previous
import contextlib
import os

import jax
import jax.numpy as jnp
from jax import lax
from jax.experimental import pallas as pl
from jax.experimental.pallas import tpu as pltpu

if os.environ.get("KERNEL_SCOPES") == "1":
    _scope = jax.named_scope
else:
    def _scope(name):
        return contextlib.nullcontext()

M, N = 8192, 1024
QROWS = M // 4
CH = 256
KQ = QROWS // CH

ZA_Z = (0, 1, 2, 3, 4)
ZA_XE = (5, 6, 7)
ZB_YB = (0, 1)
ZB_Z = (2, 3, 4, 5, 6)
ZB_XE = (7,)
XE = tuple((0, j) for j in ZA_XE) + tuple((1, j) for j in ZB_XE)


def kernel(x):
    def body(x_hbm, out_hbm, loc32, rem, sendq, out_stage,
             in_sems, bulk_sems, send_sems, xe_send, yb_send, x_recv,
             y_recv, za_recv, zb_recv, xe_recv, yb_recv, out_sems):
        my_x = lax.axis_index("x")
        my_y = lax.axis_index("y")
        my_z = lax.axis_index("z")
        px = (1 - my_x, my_y, my_z)
        py = (my_x, 1 - my_y, my_z)
        pz = (my_x, my_y, 1 - my_z)

        q_own = 2 * my_y + my_z
        q_y = 2 * (1 - my_y) + my_z
        q_za = 2 * my_y + (1 - my_z)
        q_zb = 2 * (1 - my_y) + (1 - my_z)

        def rows(q, k=0, nrows=CH):
            return pl.ds(q * QROWS + k * CH, nrows)

        in_cps = []
        for k in range(KQ):
            cp = pltpu.make_async_copy(
                x_hbm.at[rows(q_own, k), :],
                loc32.at[rows(q_own, k), :],
                in_sems.at[k],
            )
            cp.start()
            in_cps.append(cp)
        bulk_cps = []
        for i, q in enumerate((q_y, q_za, q_zb)):
            cp = pltpu.make_async_copy(
                x_hbm.at[rows(q, 0, QROWS), :],
                loc32.at[rows(q, 0, QROWS), :],
                bulk_sems.at[i],
            )
            cp.start()
            bulk_cps.append(cp)

        with _scope("barrier"):
            barrier_sem = pltpu.get_barrier_semaphore()
            for nbr in (px, py, pz):
                pl.semaphore_signal(
                    barrier_sem, inc=1, device_id=nbr,
                    device_id_type=pl.DeviceIdType.MESH,
                )
            pl.semaphore_wait(barrier_sem, 3)

        abl = os.environ.get("KERNEL_ABLATE_LINK")
        if abl in ("x", "y", "z"):
            peer = {"x": px, "y": py, "z": pz}[abl]
            nch = int(os.environ.get("KERNEL_ABLATE_NCH", "12"))
            rs = []
            for j in range(nch):
                sem = x_recv.at[j] if j < KQ else y_recv.at[j - KQ]
                r = pltpu.make_async_remote_copy(
                    src_ref=sendq.at[pl.ds((j % 12) * CH, CH), :],
                    dst_ref=rem.at[pl.ds(j * CH, CH), :],
                    send_sem=send_sems.at[0],
                    recv_sem=sem,
                    device_id=peer,
                    device_id_type=pl.DeviceIdType.MESH,
                )
                r.start()
                rs.append(r)
            for r in rs:
                r.wait_recv()
            for r in rs:
                r.wait_send()
            return

        x_rdmas = []
        for k in range(KQ):
            with _scope(f"inject#k={k}"):
                in_cps[k].wait()
                sendq[pl.ds(k * CH, CH), :] = loc32[rows(q_own, k), :].astype(
                    jnp.bfloat16)
                rdma = pltpu.make_async_remote_copy(
                    src_ref=sendq.at[pl.ds(k * CH, CH), :],
                    dst_ref=rem.at[rows(q_own, k), :],
                    send_sem=in_sems.at[k],
                    recv_sem=x_recv.at[k],
                    device_id=px,
                    device_id_type=pl.DeviceIdType.MESH,
                )
                rdma.start()
                x_rdmas.append(rdma)

        xe_rdmas = {}
        bulk_waited = [False, False]
        for idx, (qi, j) in enumerate(XE):
            with _scope(f"xe_inject#i={idx}"):
                if not bulk_waited[qi]:
                    bulk_cps[1 + qi].wait()
                    bulk_waited[qi] = True
                q = (q_za, q_zb)[qi]
                slot = pl.ds((KQ + idx) * CH, CH)
                sendq[slot, :] = loc32[rows(q, j), :].astype(jnp.bfloat16)
                rdma = pltpu.make_async_remote_copy(
                    src_ref=sendq.at[slot, :],
                    dst_ref=rem.at[rows(q, j), :],
                    send_sem=xe_send.at[idx],
                    recv_sem=xe_recv.at[idx],
                    device_id=px,
                    device_id_type=pl.DeviceIdType.MESH,
                )
                rdma.start()
                xe_rdmas[(qi, j)] = rdma

        out_cps = [None, None]
        emit_count = [0]

        def emit(q, k):
            if os.environ.get("KERNEL_ABLATE_EMIT") == "1":
                return
            i = emit_count[0]
            slot = i % 2
            if out_cps[slot] is not None:
                out_cps[slot].wait()
            out_stage[slot] = (
                loc32[rows(q, k), :].astype(jnp.bfloat16) + rem[rows(q, k), :]
            )
            cp = pltpu.make_async_copy(
                out_stage.at[slot], out_hbm.at[rows(q, k), :],
                out_sems.at[slot],
            )
            cp.start()
            out_cps[slot] = cp
            emit_count[0] += 1

        y_rdmas, za_rdmas = [], {}
        for k in range(KQ):
          with _scope(f"xhop#k={k}"):
            x_rdmas[k].wait_recv()
            ry = pltpu.make_async_remote_copy(
                src_ref=rem.at[rows(q_own, k), :],
                dst_ref=rem.at[rows(q_own, k), :],
                send_sem=send_sems.at[0],
                recv_sem=y_recv.at[k],
                device_id=py,
                device_id_type=pl.DeviceIdType.MESH,
            )
            ry.start()
            y_rdmas.append(ry)
            if k in ZA_Z:
                rza = pltpu.make_async_remote_copy(
                    src_ref=rem.at[rows(q_own, k), :],
                    dst_ref=rem.at[rows(q_own, k), :],
                    send_sem=send_sems.at[1],
                    recv_sem=za_recv.at[ZA_Z.index(k)],
                    device_id=pz,
                    device_id_type=pl.DeviceIdType.MESH,
                )
                rza.start()
                za_rdmas[k] = rza
            emit(q_own, k)

        bulk_cps[0].wait()
        zb_rdmas = {}
        for k in range(KQ):
          with _scope(f"yhop#k={k}"):
            ry_in = pltpu.make_async_remote_copy(
                src_ref=rem.at[rows(q_y, k), :],
                dst_ref=rem.at[rows(q_y, k), :],
                send_sem=send_sems.at[0],
                recv_sem=y_recv.at[k],
                device_id=py,
                device_id_type=pl.DeviceIdType.MESH,
            )
            ry_in.wait_recv()
            if k in ZB_Z:
                rzb = pltpu.make_async_remote_copy(
                    src_ref=rem.at[rows(q_y, k), :],
                    dst_ref=rem.at[rows(q_y, k), :],
                    send_sem=send_sems.at[2],
                    recv_sem=zb_recv.at[ZB_Z.index(k)],
                    device_id=pz,
                    device_id_type=pl.DeviceIdType.MESH,
                )
                rzb.start()
                zb_rdmas[k] = rzb
            emit(q_y, k)

        yb_rdmas = []
        for k in ZA_Z:
            with _scope(f"za#k={k}"):
                za_rdmas[k].wait_recv()
                if k in ZB_YB:
                    ryb = pltpu.make_async_remote_copy(
                        src_ref=rem.at[rows(q_za, k), :],
                        dst_ref=rem.at[rows(q_za, k), :],
                        send_sem=yb_send.at[k],
                        recv_sem=yb_recv.at[k],
                        device_id=py,
                        device_id_type=pl.DeviceIdType.MESH,
                    )
                    ryb.start()
                    yb_rdmas.append(ryb)
                emit(q_za, k)
        with _scope("za_xe"):
            for j in ZA_XE:
                xe_rdmas[(0, j)].wait_recv()
                emit(q_za, j)

        for k in ZB_YB:
            with _scope(f"yb#k={k}"):
                ryb_in = pltpu.make_async_remote_copy(
                    src_ref=rem.at[rows(q_zb, k), :],
                    dst_ref=rem.at[rows(q_zb, k), :],
                    send_sem=yb_send.at[k],
                    recv_sem=yb_recv.at[k],
                    device_id=py,
                    device_id_type=pl.DeviceIdType.MESH,
                )
                ryb_in.wait_recv()
                emit(q_zb, k)
        for k in ZB_Z:
            with _scope(f"zb#k={k}"):
                zb_rdmas[k].wait_recv()
                emit(q_zb, k)
        with _scope("zb_xe"):
            for j in ZB_XE:
                xe_rdmas[(1, j)].wait_recv()
                emit(q_zb, j)

        with _scope("drain"):
            for cp in out_cps:
                if cp is not None:
                    cp.wait()
            for k in range(KQ):
                x_rdmas[k].wait_send()
                y_rdmas[k].wait_send()
            for r in za_rdmas.values():
                r.wait_send()
            for r in zb_rdmas.values():
                r.wait_send()
            for r in xe_rdmas.values():
                r.wait_send()
            for r in yb_rdmas:
                r.wait_send()

    return pl.pallas_call(
        body,
        out_shape=jax.ShapeDtypeStruct((M, N), jnp.bfloat16),
        in_specs=[pl.BlockSpec(memory_space=pl.ANY)],
        out_specs=pl.BlockSpec(memory_space=pl.ANY),
        scratch_shapes=[
            pltpu.VMEM((M, N), jnp.float32),
            pltpu.VMEM((M, N), jnp.bfloat16),
            pltpu.VMEM((QROWS + len(XE) * CH, N), jnp.bfloat16),
            pltpu.VMEM((2, CH, N), jnp.bfloat16),
            pltpu.SemaphoreType.DMA((KQ,)),
            pltpu.SemaphoreType.DMA((3,)),
            pltpu.SemaphoreType.DMA((3,)),
            pltpu.SemaphoreType.DMA((len(XE),)),
            pltpu.SemaphoreType.DMA((len(ZB_YB),)),
            pltpu.SemaphoreType.DMA((KQ,)),
            pltpu.SemaphoreType.DMA((KQ,)),
            pltpu.SemaphoreType.DMA((len(ZA_Z),)),
            pltpu.SemaphoreType.DMA((len(ZB_Z),)),
            pltpu.SemaphoreType.DMA((len(XE),)),
            pltpu.SemaphoreType.DMA((len(ZB_YB),)),
            pltpu.SemaphoreType.DMA((2,)),
        ],
        compiler_params=pltpu.CompilerParams(
            collective_id=0, vmem_limit_bytes=60 * 1024 * 1024),
    )(x)


# device time: 96411 ns/iter; 1.1648x vs baseline; 1.0019x over previous
import contextlib
import os

import jax
import jax.numpy as jnp
from jax import lax
from jax.experimental import pallas as pl
from jax.experimental.pallas import tpu as pltpu

if os.environ.get("KERNEL_SCOPES") == "1":
    _scope = jax.named_scope
else:
    def _scope(name):
        return contextlib.nullcontext()

M, N = 8192, 1024
QROWS = M // 4
CH = 256
KQ = QROWS // CH

ZA_Z = (0, 1, 2, 3, 4)
ZA_XE = (5, 6, 7)
ZB_YB = (0, 1)
ZB_Z = (2, 3, 4, 5, 6)
ZB_XE = (7,)
XE = tuple((0, j) for j in ZA_XE) + tuple((1, j) for j in ZB_XE)


def kernel(x):
    def body(x_hbm, out_hbm, loc32, rem, sendq, out_stage,
             in_sems, bulk_sems, send_sems, xe_send, yb_send, x_recv,
             y_recv, za_recv, zb_recv, xe_recv, yb_recv, out_sems):
        my_x = lax.axis_index("x")
        my_y = lax.axis_index("y")
        my_z = lax.axis_index("z")
        px = (1 - my_x, my_y, my_z)
        py = (my_x, 1 - my_y, my_z)
        pz = (my_x, my_y, 1 - my_z)

        q_own = 2 * my_y + my_z
        q_y = 2 * (1 - my_y) + my_z
        q_za = 2 * my_y + (1 - my_z)
        q_zb = 2 * (1 - my_y) + (1 - my_z)

        def rows(q, k=0, nrows=CH):
            return pl.ds(q * QROWS + k * CH, nrows)

        in_cps = []
        for k in range(KQ):
            cp = pltpu.make_async_copy(
                x_hbm.at[rows(q_own, k), :],
                loc32.at[rows(q_own, k), :],
                in_sems.at[k],
            )
            cp.start()
            in_cps.append(cp)
        bulk_cps = []
        for i, q in enumerate((q_za, q_zb, q_y)):
            cp = pltpu.make_async_copy(
                x_hbm.at[rows(q, 0, QROWS), :],
                loc32.at[rows(q, 0, QROWS), :],
                bulk_sems.at[i],
            )
            cp.start()
            bulk_cps.append(cp)

        with _scope("barrier"):
            barrier_sem = pltpu.get_barrier_semaphore()
            for nbr in (px, py, pz):
                pl.semaphore_signal(
                    barrier_sem, inc=1, device_id=nbr,
                    device_id_type=pl.DeviceIdType.MESH,
                )
            pl.semaphore_wait(barrier_sem, 3)

        abl = os.environ.get("KERNEL_ABLATE_LINK")
        if abl in ("x", "y", "z"):
            peer = {"x": px, "y": py, "z": pz}[abl]
            nch = int(os.environ.get("KERNEL_ABLATE_NCH", "12"))
            rs = []
            for j in range(nch):
                sem = x_recv.at[j] if j < KQ else y_recv.at[j - KQ]
                r = pltpu.make_async_remote_copy(
                    src_ref=sendq.at[pl.ds((j % 12) * CH, CH), :],
                    dst_ref=rem.at[pl.ds(j * CH, CH), :],
                    send_sem=send_sems.at[0],
                    recv_sem=sem,
                    device_id=peer,
                    device_id_type=pl.DeviceIdType.MESH,
                )
                r.start()
                rs.append(r)
            for r in rs:
                r.wait_recv()
            for r in rs:
                r.wait_send()
            return

        x_rdmas = []
        for k in range(KQ):
            with _scope(f"inject#k={k}"):
                in_cps[k].wait()
                sendq[pl.ds(k * CH, CH), :] = loc32[rows(q_own, k), :].astype(
                    jnp.bfloat16)
                rdma = pltpu.make_async_remote_copy(
                    src_ref=sendq.at[pl.ds(k * CH, CH), :],
                    dst_ref=rem.at[rows(q_own, k), :],
                    send_sem=in_sems.at[k],
                    recv_sem=x_recv.at[k],
                    device_id=px,
                    device_id_type=pl.DeviceIdType.MESH,
                )
                rdma.start()
                x_rdmas.append(rdma)

        xe_rdmas = {}

        def xe_inject(qi):
            bulk_cps[qi].wait()
            q = (q_za, q_zb)[qi]
            for j in (ZA_XE, ZB_XE)[qi]:
                idx = XE.index((qi, j))
                slot = pl.ds((KQ + idx) * CH, CH)
                sendq[slot, :] = loc32[rows(q, j), :].astype(jnp.bfloat16)
                rdma = pltpu.make_async_remote_copy(
                    src_ref=sendq.at[slot, :],
                    dst_ref=rem.at[rows(q, j), :],
                    send_sem=xe_send.at[idx],
                    recv_sem=xe_recv.at[idx],
                    device_id=px,
                    device_id_type=pl.DeviceIdType.MESH,
                )
                rdma.start()
                xe_rdmas[(qi, j)] = rdma

        out_cps = [None, None]
        emit_count = [0]

        def emit(q, k):
            if os.environ.get("KERNEL_ABLATE_EMIT") == "1":
                return
            i = emit_count[0]
            slot = i % 2
            if out_cps[slot] is not None:
                out_cps[slot].wait()
            out_stage[slot] = (
                loc32[rows(q, k), :].astype(jnp.bfloat16) + rem[rows(q, k), :]
            )
            cp = pltpu.make_async_copy(
                out_stage.at[slot], out_hbm.at[rows(q, k), :],
                out_sems.at[slot],
            )
            cp.start()
            out_cps[slot] = cp
            emit_count[0] += 1

        y_rdmas, za_rdmas = [], {}
        for k in range(KQ):
          with _scope(f"xhop#k={k}"):
            x_rdmas[k].wait_recv()
            ry = pltpu.make_async_remote_copy(
                src_ref=rem.at[rows(q_own, k), :],
                dst_ref=rem.at[rows(q_own, k), :],
                send_sem=send_sems.at[0],
                recv_sem=y_recv.at[k],
                device_id=py,
                device_id_type=pl.DeviceIdType.MESH,
            )
            ry.start()
            y_rdmas.append(ry)
            if k in ZA_Z:
                rza = pltpu.make_async_remote_copy(
                    src_ref=rem.at[rows(q_own, k), :],
                    dst_ref=rem.at[rows(q_own, k), :],
                    send_sem=send_sems.at[1],
                    recv_sem=za_recv.at[ZA_Z.index(k)],
                    device_id=pz,
                    device_id_type=pl.DeviceIdType.MESH,
                )
                rza.start()
                za_rdmas[k] = rza
            emit(q_own, k)
            if k == 4:
                xe_inject(0)
            elif k == 5:
                xe_inject(1)

        bulk_cps[2].wait()
        zb_rdmas = {}
        for k in range(KQ):
          with _scope(f"yhop#k={k}"):
            ry_in = pltpu.make_async_remote_copy(
                src_ref=rem.at[rows(q_y, k), :],
                dst_ref=rem.at[rows(q_y, k), :],
                send_sem=send_sems.at[0],
                recv_sem=y_recv.at[k],
                device_id=py,
                device_id_type=pl.DeviceIdType.MESH,
            )
            ry_in.wait_recv()
            if k in ZB_Z:
                rzb = pltpu.make_async_remote_copy(
                    src_ref=rem.at[rows(q_y, k), :],
                    dst_ref=rem.at[rows(q_y, k), :],
                    send_sem=send_sems.at[2],
                    recv_sem=zb_recv.at[ZB_Z.index(k)],
                    device_id=pz,
                    device_id_type=pl.DeviceIdType.MESH,
                )
                rzb.start()
                zb_rdmas[k] = rzb
            emit(q_y, k)

        yb_rdmas = []
        for k in ZA_Z:
            with _scope(f"za#k={k}"):
                za_rdmas[k].wait_recv()
                if k in ZB_YB:
                    ryb = pltpu.make_async_remote_copy(
                        src_ref=rem.at[rows(q_za, k), :],
                        dst_ref=rem.at[rows(q_za, k), :],
                        send_sem=yb_send.at[k],
                        recv_sem=yb_recv.at[k],
                        device_id=py,
                        device_id_type=pl.DeviceIdType.MESH,
                    )
                    ryb.start()
                    yb_rdmas.append(ryb)
                emit(q_za, k)
        with _scope("za_xe"):
            for j in ZA_XE:
                xe_rdmas[(0, j)].wait_recv()
                emit(q_za, j)

        for k in ZB_YB:
            with _scope(f"yb#k={k}"):
                ryb_in = pltpu.make_async_remote_copy(
                    src_ref=rem.at[rows(q_zb, k), :],
                    dst_ref=rem.at[rows(q_zb, k), :],
                    send_sem=yb_send.at[k],
                    recv_sem=yb_recv.at[k],
                    device_id=py,
                    device_id_type=pl.DeviceIdType.MESH,
                )
                ryb_in.wait_recv()
                emit(q_zb, k)
        for k in ZB_Z:
            with _scope(f"zb#k={k}"):
                zb_rdmas[k].wait_recv()
                emit(q_zb, k)
        with _scope("zb_xe"):
            for j in ZB_XE:
                xe_rdmas[(1, j)].wait_recv()
                emit(q_zb, j)

        with _scope("drain"):
            for cp in out_cps:
                if cp is not None:
                    cp.wait()
            for k in range(KQ):
                x_rdmas[k].wait_send()
                y_rdmas[k].wait_send()
            for r in za_rdmas.values():
                r.wait_send()
            for r in zb_rdmas.values():
                r.wait_send()
            for r in xe_rdmas.values():
                r.wait_send()
            for r in yb_rdmas:
                r.wait_send()

    return pl.pallas_call(
        body,
        out_shape=jax.ShapeDtypeStruct((M, N), jnp.bfloat16),
        in_specs=[pl.BlockSpec(memory_space=pl.ANY)],
        out_specs=pl.BlockSpec(memory_space=pl.ANY),
        scratch_shapes=[
            pltpu.VMEM((M, N), jnp.float32),
            pltpu.VMEM((M, N), jnp.bfloat16),
            pltpu.VMEM((QROWS + len(XE) * CH, N), jnp.bfloat16),
            pltpu.VMEM((2, CH, N), jnp.bfloat16),
            pltpu.SemaphoreType.DMA((KQ,)),
            pltpu.SemaphoreType.DMA((3,)),
            pltpu.SemaphoreType.DMA((3,)),
            pltpu.SemaphoreType.DMA((len(XE),)),
            pltpu.SemaphoreType.DMA((len(ZB_YB),)),
            pltpu.SemaphoreType.DMA((KQ,)),
            pltpu.SemaphoreType.DMA((KQ,)),
            pltpu.SemaphoreType.DMA((len(ZA_Z),)),
            pltpu.SemaphoreType.DMA((len(ZB_Z),)),
            pltpu.SemaphoreType.DMA((len(XE),)),
            pltpu.SemaphoreType.DMA((len(ZB_YB),)),
            pltpu.SemaphoreType.DMA((2,)),
        ],
        compiler_params=pltpu.CompilerParams(
            collective_id=0, vmem_limit_bytes=60 * 1024 * 1024),
    )(x)
